# Initial kernel scaffold; baseline (speedup 1.0000x reference)
import jax
import jax.numpy as jnp
from jax.experimental import pallas as pl


def kernel(x, inds):
    raise NotImplementedError("write your pallas kernel here")


# speedup vs baseline: 3.2130x; 3.2130x over previous
import functools

import jax
import jax.numpy as jnp
from jax import lax
from jax.experimental import pallas as pl
from jax.experimental.pallas import tpu as pltpu
from jax.experimental.pallas import tpu_sc as plsc

H = 496
W = 432
B = 4
C = 64
P = 12000

NC = 2
NS = 16
NW = NC * NS
WPB = NW // B
RPW = H // WPB
L = 16

CHUNK = 1200
NCHUNK = P // CHUNK
WMAX = 448
NGRP = WMAX // L
SENT = 0x40000000


def _tc_transpose(x):
    def body(x_ref, o_ref):
        o_ref[...] = x_ref[0].T

    return pl.pallas_call(
        body,
        grid=(B,),
        in_specs=[pl.BlockSpec((1, C, P), lambda b: (b, 0, 0))],
        out_specs=pl.BlockSpec((P, C), lambda b: (b, 0)),
        out_shape=jax.ShapeDtypeStruct((B * P, C), jnp.float32),
    )(x)


def _sc_scatter(flag, xi, yi, xt):
    mesh = plsc.VectorSubcoreMesh(core_axis_name="c", subcore_axis_name="s")

    @functools.partial(
        pl.kernel,
        out_type=jax.ShapeDtypeStruct((B, C, H, W), jnp.float32),
        mesh=mesh,
        compiler_params=pltpu.CompilerParams(
            use_tc_tiling_on_sc=False, needs_layout_passes=False),
        scratch_types=dict(
            m_map=pltpu.VMEM((RPW * W,), jnp.int32),
            fbuf=pltpu.VMEM((CHUNK,), jnp.int32),
            xbuf=pltpu.VMEM((CHUNK,), jnp.int32),
            ybuf=pltpu.VMEM((CHUNK,), jnp.int32),
            slab0=pltpu.VMEM((C, W), jnp.float32),
            slab1=pltpu.VMEM((C, W), jnp.float32),
            gbuf=pltpu.VMEM((WMAX, C), jnp.float32),
            idxbuf=pltpu.VMEM((WMAX,), jnp.int32),
            colbuf=pltpu.VMEM((2 * WMAX,), jnp.int32),
            scr=pltpu.VMEM((L,), jnp.int32),
            gsem=pltpu.SemaphoreType.DMA,
            osem0=pltpu.SemaphoreType.DMA,
            osem1=pltpu.SemaphoreType.DMA,
        ),
    )
    def kern(flag_hbm, xi_hbm, yi_hbm, xt_hbm, out_hbm, m_map, fbuf, xbuf,
             ybuf, slab0, slab1, gbuf, idxbuf, colbuf, scr, gsem, osem0,
             osem1):
        wid = lax.axis_index("s") * NC + lax.axis_index("c")
        b = wid // WPB
        octant = wid % WPB
        y0 = octant * RPW

        lane = lax.iota(jnp.int32, L)
        zeros_f = jnp.zeros((L,), jnp.float32)
        zeros_i = jnp.zeros((L,), jnp.int32)
        ones_i = jnp.ones((L,), jnp.int32)

        def initm(i, carry):
            m_map[pl.ds(i * L, L)] = jnp.full((L,), -1, jnp.int32)
            return carry

        lax.fori_loop(0, RPW * W // L, initm, 0)

        def initslab(c, carry):
            for slab in (slab0, slab1):
                for g in range(W // L):
                    slab.at[c][pl.ds(g * L, L)] = zeros_f
            return carry

        lax.fori_loop(0, C, initslab, 0)

        def initidx(i, carry):
            idxbuf[pl.ds(i * L, L)] = zeros_i
            return carry

        lax.fori_loop(0, NGRP, initidx, 0)

        for k in range(NCHUNK):
            pltpu.sync_copy(flag_hbm.at[b, pl.ds(k * CHUNK, CHUNK)], fbuf)
            pltpu.sync_copy(xi_hbm.at[b, pl.ds(k * CHUNK, CHUNK)], xbuf)
            pltpu.sync_copy(yi_hbm.at[b, pl.ds(k * CHUNK, CHUNK)], ybuf)

            def scan_pillars(g, carry, k=k):
                f = fbuf[pl.ds(g * L, L)]
                xx = xbuf[pl.ds(g * L, L)]
                yy = ybuf[pl.ds(g * L, L)]
                valid = (f == 1) & (yy >= y0) & (yy < y0 + RPW)
                loc = (yy - y0) * W + xx
                key = jnp.where(valid, loc * L + lane, SENT + lane)
                skey, _ = plsc.sort_key_val(key, key)
                scell = skey >> 4
                nxt = lax.gather(
                    scell, jnp.minimum(lane + 1, L - 1)[:, None],
                    lax.GatherDimensionNumbers(
                        offset_dims=(), collapsed_slice_dims=(0,),
                        start_index_map=(0,)),
                    slice_sizes=(1,),
                    mode=lax.GatherScatterMode.PROMISE_IN_BOUNDS)
                keep_sorted = (scell != nxt) | (lane == L - 1)
                slane = skey & (L - 1)
                scr[pl.ds(0, L)] = ones_i
                plsc.store_scatter(scr, [slane], zeros_i,
                                   mask=jnp.logical_not(keep_sorted))
                keepvec = scr[pl.ds(0, L)]
                m = valid & (keepvec == 1)
                pid = jnp.full((L,), b * P + k * CHUNK + g * L,
                               jnp.int32) + lane
                plsc.store_scatter(m_map, [loc], pid, mask=m)
                return carry

            lax.fori_loop(0, CHUNK // L, scan_pillars, 0)

        def do_row(i, par, kprev, slab, osem):
            row = i * 2 + par
            @pl.when(i > 0)
            def _():
                pltpu.make_async_copy(
                    slab, out_hbm.at[b, :, y0 + row - 2, :], osem).wait()
                for g in range(NGRP):
                    @pl.when(kprev > g * L)
                    def _(g=g):
                        colv = colbuf[pl.ds(par * WMAX + g * L, L)]
                        msk = lane + g * L < kprev

                        def clear_c(c, carry):
                            plsc.store_scatter(slab.at[c], [colv], zeros_f,
                                               mask=msk)
                            return carry

                        lax.fori_loop(0, C, clear_c, 0)

            def scan_row(g, j):
                mrow = m_map[pl.ds(row * W + g * L, L)]
                msk = mrow >= 0
                cnt = jnp.sum(msk.astype(jnp.int32))
                ranks = plsc.cumsum(ones_i, mask=msk)
                pos = j + ranks - 1
                plsc.store_scatter(colbuf, [pos + par * WMAX],
                                   lane + g * L, mask=msk)
                plsc.store_scatter(idxbuf, [pos], mrow, mask=msk)
                return j + cnt

            nwin = lax.fori_loop(0, W // L, scan_row, 0)

            for g in range(NGRP):
                @pl.when(nwin > g * L)
                def _(g=g):
                    pltpu.async_copy(
                        xt_hbm.at[idxbuf.at[pl.ds(g * L, L)]],
                        gbuf.at[pl.ds(g * L, L)], gsem)
            for g in range(NGRP):
                @pl.when(nwin > g * L)
                def _(g=g):
                    pltpu.make_async_copy(
                        xt_hbm.at[idxbuf.at[pl.ds(g * L, L)]],
                        gbuf.at[pl.ds(g * L, L)], gsem).wait()

            for g in range(NGRP):
                @pl.when(nwin > g * L)
                def _(g=g):
                    colv = colbuf[pl.ds(par * WMAX + g * L, L)]
                    wv = lane + g * L
                    msk = wv < nwin

                    def paint_c(c, carry):
                        vals = plsc.load_gather(
                            gbuf, [wv, jnp.full((L,), c, jnp.int32)],
                            mask=msk)
                        plsc.store_scatter(slab.at[c], [colv], vals,
                                           mask=msk)
                        return carry

                    lax.fori_loop(0, C, paint_c, 0)

            pltpu.async_copy(slab, out_hbm.at[b, :, y0 + row, :], osem)
            return nwin

        def row_pair(i, carry):
            kprev0, kprev1 = carry
            k0 = do_row(i, 0, kprev0, slab0, osem0)
            k1 = do_row(i, 1, kprev1, slab1, osem1)
            return (k0, k1)

        lax.fori_loop(0, RPW // 2, row_pair, (0, 0))

        pltpu.make_async_copy(
            slab0, out_hbm.at[b, :, y0 + RPW - 2, :], osem0).wait()
        pltpu.make_async_copy(
            slab1, out_hbm.at[b, :, y0 + RPW - 1, :], osem1).wait()

    return kern(flag, xi, yi, xt)


def kernel(x, inds):
    flag = inds[..., 0].astype(jnp.int32)
    xi = inds[..., 1].astype(jnp.int32)
    yi = inds[..., 2].astype(jnp.int32)
    xt = _tc_transpose(x)
    return _sc_scatter(flag, xi, yi, xt)
